# baseline (speedup 1.0000x reference)
import functools

import jax
import jax.numpy as jnp
from jax import lax
from jax.experimental import pallas as pl
from jax.experimental.pallas import tpu as pltpu
from jax.experimental.pallas import tpu_sc as plsc

N_NODES = 10000
N_EDGES = 320000
D = 128
HID = 128
EAD = 16

BR = 1000
SLOTS = 8


def _edge_mlp_body(attr2_ref, w1b_ref, b1_ref, w2_ref, b2_ref, wl_ref,
                   bl_ref, out_ref):
  attr2 = attr2_ref[...]
  a2 = jnp.dot(attr2, w1b_ref[...], preferred_element_type=jnp.float32)
  for j in range(SLOTS):
    h = jnp.maximum(a2[:, j * HID:(j + 1) * HID] + b1_ref[...], 0.0)
    e = jnp.dot(h, w2_ref[...], preferred_element_type=jnp.float32)
    pol = jnp.clip(attr2[:, j * EAD:j * EAD + 1], 0.0, 1.0) + 0.01
    g = (e + b2_ref[...]) * pol
    o = jnp.dot(g, wl_ref[...], preferred_element_type=jnp.float32)
    out_ref[j, :, :] = o + bl_ref[...]


def _edge_mlp(attr2, w1b, b1, w2, b2, wlin, blin):
  n_blk = N_EDGES // (SLOTS * BR)
  wspec = lambda shape: pl.BlockSpec(shape, lambda i: (0, 0))
  return pl.pallas_call(
      _edge_mlp_body,
      grid=(n_blk,),
      in_specs=[
          pl.BlockSpec((BR, SLOTS * EAD), lambda i: (i, 0)),
          wspec((SLOTS * EAD, SLOTS * HID)),
          wspec((1, HID)),
          wspec((HID, HID)),
          wspec((1, HID)),
          wspec((HID, D)),
          wspec((1, D)),
      ],
      out_specs=pl.BlockSpec((SLOTS, BR, D), lambda i: (0, i, 0)),
      out_shape=jax.ShapeDtypeStruct((SLOTS, N_EDGES // SLOTS, D),
                                     jnp.float32),
  )(attr2, w1b, b1, w2, b2, wlin, blin)


_INFO = plsc.get_sparse_core_info()
NC = _INFO.num_cores
NS = _INFO.num_subcores
NW = NC * NS
EPW = N_EDGES // NW
KB = 80
NB = EPW // KB
NPAD = 10240
RPS = NPAD // NS


def _sc_body(x_hbm, src_hbm, dst_hbm, ea_hbm, zeros_hbm, out_hbm,
             si, di, xb, eb, ssp, sdp, sxb, seb, agg_sh):
  c = lax.axis_index("c")
  s = lax.axis_index("s")
  wid = c * NS + s
  j = wid // 4
  mbase = (wid % 4) * (NB * KB)
  ebase = j * (N_EDGES // SLOTS) + mbase

  pltpu.sync_copy(zeros_hbm.at[pl.ds(s * RPS, RPS)],
                  agg_sh.at[pl.ds(s * RPS, RPS)])
  plsc.subcore_barrier()

  def start_idx(b, k):
    base = ebase + b * KB
    pltpu.async_copy(src_hbm.at[pl.ds(base, KB)], si[k], ssp[k])
    pltpu.async_copy(dst_hbm.at[pl.ds(base, KB)], di[k], sdp[k])

  def wait_idx(k):
    pltpu.make_async_copy(src_hbm.at[pl.ds(0, KB)], si[k], ssp[k]).wait()
    pltpu.make_async_copy(dst_hbm.at[pl.ds(0, KB)], di[k], sdp[k]).wait()

  def start_data(b, k):
    pltpu.async_copy(x_hbm.at[si[k]], xb[k], sxb[k])
    pltpu.async_copy(ea_hbm.at[pl.ds(ebase + b * KB, KB)], eb[k], seb[k])

  def wait_data(k):
    pltpu.make_async_copy(x_hbm.at[pl.ds(0, KB)], xb[k], sxb[k]).wait()
    pltpu.make_async_copy(ea_hbm.at[pl.ds(0, KB)], eb[k], seb[k]).wait()

  def compute_scatter(k):
    xbuf, ebuf = xb[k], eb[k]

    @plsc.parallel_loop(0, KB, unroll=2)
    def _(i):
      for j in range(D // 16):
        a = xbuf[i, pl.ds(j * 16, 16)]
        v = ebuf[i, pl.ds(j * 16, 16)]
        ebuf[i, pl.ds(j * 16, 16)] = jnp.maximum(a + v, 0.0)

    pltpu.sync_copy(ebuf, agg_sh.at[di[k]], add=True)

  start_idx(0, 0)
  wait_idx(0)
  start_data(0, 0)
  start_idx(1, 1)

  def step(b, k):
    @pl.when(b + 1 < NB)
    def _():
      wait_idx(1 - k)
      start_data(b + 1, 1 - k)

    wait_data(k)
    compute_scatter(k)

    @pl.when(b + 2 < NB)
    def _():
      start_idx(b + 2, k)

  def pair(i, carry):
    step(2 * i, 0)

    @pl.when(2 * i + 1 < NB)
    def _():
      step(2 * i + 1, 1)

    return carry

  lax.fori_loop(0, (NB + 1) // 2, pair, 0)

  plsc.subcore_barrier()
  pltpu.sync_copy(agg_sh.at[pl.ds(s * RPS, RPS)],
                  out_hbm.at[c].at[pl.ds(s * RPS, RPS)])


def _sc_aggregate(x, src, dst, ea, zeros):
  mesh = plsc.VectorSubcoreMesh(core_axis_name="c", subcore_axis_name="s")
  f = pl.kernel(
      _sc_body,
      out_type=jax.ShapeDtypeStruct((NC, NPAD, D), jnp.float32),
      mesh=mesh,
      scratch_types=[
          [pltpu.VMEM((KB,), jnp.int32)] * 2,
          [pltpu.VMEM((KB,), jnp.int32)] * 2,
          [pltpu.VMEM((KB, D), jnp.float32)] * 2,
          [pltpu.VMEM((KB, D), jnp.float32)] * 2,
          [pltpu.SemaphoreType.DMA] * 2,
          [pltpu.SemaphoreType.DMA] * 2,
          [pltpu.SemaphoreType.DMA] * 2,
          [pltpu.SemaphoreType.DMA] * 2,
          pltpu.VMEM_SHARED((NPAD, D), jnp.float32),
      ],
  )
  return f(x, src, dst, ea, zeros)


BN = 2000


def _node_mlp_body(p_ref, x_ref, wa_ref, ba_ref, g_ref, bt_ref, wb_ref,
                   bb_ref, out_ref):
  out = p_ref[0] + p_ref[1] + x_ref[...]
  h2 = jnp.dot(out, wa_ref[...], preferred_element_type=jnp.float32)
  h2 = h2 + ba_ref[...]
  mu = jnp.mean(h2, axis=-1, keepdims=True)
  d = h2 - mu
  var = jnp.mean(d * d, axis=-1, keepdims=True)
  h2 = d * lax.rsqrt(var + 1e-5) * g_ref[...] + bt_ref[...]
  h2 = jnp.maximum(h2, 0.0)
  o = jnp.dot(h2, wb_ref[...], preferred_element_type=jnp.float32)
  out_ref[...] = o + bb_ref[...]


def _node_mlp(partials, x, wa, ba, ln_g, ln_b, wb, bb):
  n_blk = N_NODES // BN
  wspec = lambda shape: pl.BlockSpec(shape, lambda i: (0, 0))
  return pl.pallas_call(
      _node_mlp_body,
      grid=(n_blk,),
      in_specs=[
          pl.BlockSpec((NC, BN, D), lambda i: (0, i, 0)),
          pl.BlockSpec((BN, D), lambda i: (i, 0)),
          wspec((D, D)),
          wspec((1, D)),
          wspec((1, D)),
          wspec((1, D)),
          wspec((D, D)),
          wspec((1, D)),
      ],
      out_specs=pl.BlockSpec((BN, D), lambda i: (i, 0)),
      out_shape=jax.ShapeDtypeStruct((N_NODES, D), jnp.float32),
  )(partials, x, wa, ba, ln_g, ln_b, wb, bb)


def kernel(x, edge_index, edge_attr, W1, b1, W2, b2, Wlin, blin, Wa, ba,
           ln_g, ln_b, Wb, bb):
  w1p = jnp.concatenate([jnp.zeros((1, HID), jnp.float32), W1], axis=0)
  w1b = jax.scipy.linalg.block_diag(*([w1p] * SLOTS))
  attr2 = edge_attr.reshape(N_EDGES // SLOTS, SLOTS * EAD)
  ea3 = _edge_mlp(attr2, w1b, b1[None, :], W2, b2[None, :], Wlin,
                  blin[None, :])
  ea = ea3.reshape(N_EDGES, D)
  ei = (edge_index.reshape(2, N_EDGES // SLOTS, SLOTS)
        .transpose(0, 2, 1).reshape(2, N_EDGES))
  zeros = jnp.zeros((NPAD, D), jnp.float32)
  partials = _sc_aggregate(x, ei[0], ei[1], ea, zeros)
  return _node_mlp(partials, x, Wa, ba[None, :], ln_g[None, :],
                   ln_b[None, :], Wb, bb[None, :])
